# baseline (speedup 1.0000x reference)
import functools

import numpy as np
import jax
import jax.numpy as jnp
from jax import lax
from jax.experimental import pallas as pl
from jax.experimental.pallas import tpu as pltpu
from jax.experimental.pallas import tpu_sc as plsc

N = 10000
E = 160000
NP = 10240
EP = 163840
MUL0 = 16
MUL1 = 8
CUTOFF = 10.0
NBASIS = 16
NUM_NEIGHBORS = 20
C = float(np.sqrt(0.5))
GAMMA = float(NBASIS / CUTOFF)
SH0 = float(1.0 / np.sqrt(NUM_NEIGHBORS))
SH1C = float(np.sqrt(3.0) / np.sqrt(NUM_NEIGHBORS))
INV_SQRT3 = float(1.0 / np.sqrt(3.0))
INV_SQRT24 = float(1.0 / np.sqrt(MUL0 + MUL1))

CHUNK = 128
NCHUNKS = EP // CHUNK
NTILES = 32
CPT = NCHUNKS // NTILES
GRP = 10
NGRP = CPT // GRP
ROWS_PT = NP // 16
GGRP = 10
G1_C0, G1_C1 = 40, 40
G2_C0, G2_C1 = 40, 40
CPTMAX = 40

BE = 2048
GE = EP // BE

CSTEP = float(CUTOFF / (NBASIS - 1))

_PERM = np.concatenate(
    [np.concatenate([np.arange(u * 24, (u + 1) * 24),
                     384 + np.arange(u * 8, (u + 1) * 8)]) for u in range(16)]
)


def _swish(x):
    return x * jax.nn.sigmoid(x)


def _swish16(x):
    return (x * jax.nn.sigmoid(x)).astype(jnp.bfloat16)


def _np_r16():
    r = np.zeros((16, 512), np.float32)
    for u in range(16):
        r[u, u * 32:(u + 1) * 32] = 1.0
    return r


def _np_t32():
    t = np.zeros((32, 512), np.float32)
    for k in range(32):
        t[k, k::32] = 1.0
    return t


def _np_vrep():
    a = np.zeros((8, 24), np.float32)
    for v in range(8):
        for m in range(3):
            a[v, 8 * m + v] = 1.0
    return a


def _np_srep():
    a = np.zeros((3, 24), np.float32)
    for m in range(3):
        a[m, 8 * m:8 * m + 8] = 1.0
    return a


def _np_sexp48():
    a = np.zeros((3, 48), np.float32)
    for m in range(3):
        a[m, 16 + 8 * m:24 + 8 * m] = 1.0
    return a


def _np_msum48():
    a = np.zeros((48, 8), np.float32)
    for m in range(3):
        for v in range(8):
            a[16 + 8 * m + v, v] = 1.0
    return a


_R16 = _np_r16()
_T32 = _np_t32()
_VREP = _np_vrep()
_SREP = _np_srep()
_SEXP48 = _np_sexp48()
_MSUM48 = _np_msum48()
_ONES24 = np.ones((24, 1), np.float32)


def _embed_body(z_ref, emb_ref, posp_ref, ts_ref):
    z = z_ref[...]
    k = lax.broadcasted_iota(jnp.int32, (NP, 100), 1)
    oh = (z == k).astype(jnp.float32)
    h0 = jnp.dot(oh, emb_ref[...], preferred_element_type=jnp.float32)
    ts_ref[...] = jnp.concatenate([posp_ref[...], h0], axis=1)


_embed = pl.pallas_call(
    _embed_body,
    out_shape=jax.ShapeDtypeStruct((NP, 32), jnp.float32),
)


def _gather1_body(src2, dst2, ts_hbm, posp_hbm, edat_hbm,
                  idxs_v, idxd_v, osrc_v, odst_v, sem):
    c = lax.axis_index("c")
    s = lax.axis_index("s")
    cbase = (s * 2 + c) * CPT
    ngrp = CPT // GGRP
    pltpu.sync_copy(src2.at[pl.ds(cbase, CPTMAX)], idxs_v)
    pltpu.sync_copy(dst2.at[pl.ds(cbase, CPTMAX)], idxd_v)

    def body(g, _):
        descs = []
        for j in range(GGRP):
            jj = g * GGRP + j
            descs.append(pltpu.async_copy(
                ts_hbm.at[idxs_v.at[jj]],
                osrc_v.at[pl.ds(j * CHUNK, CHUNK)], sem))
            descs.append(pltpu.async_copy(
                posp_hbm.at[idxd_v.at[jj]],
                odst_v.at[pl.ds(j * CHUNK, CHUNK)], sem))
        for d in descs:
            d.wait()
        ebase = (cbase + g * GGRP) * CHUNK
        pltpu.sync_copy(osrc_v,
                        edat_hbm.at[pl.ds(ebase, GGRP * CHUNK), pl.ds(0, 32)])
        pltpu.sync_copy(odst_v,
                        edat_hbm.at[pl.ds(ebase, GGRP * CHUNK), pl.ds(32, 16)])
        return ()

    lax.fori_loop(0, ngrp, body, ())


@functools.cache
def _gather1_kernel():
    return pl.kernel(
        _gather1_body,
        out_type=jax.ShapeDtypeStruct((EP, 128), jnp.float32),
        mesh=plsc.VectorSubcoreMesh(core_axis_name="c", subcore_axis_name="s"),
        compiler_params=pltpu.CompilerParams(use_tc_tiling_on_sc=False),
        scratch_types=(pltpu.VMEM((CPTMAX, CHUNK), jnp.int32),
                       pltpu.VMEM((CPTMAX, CHUNK), jnp.int32),
                       pltpu.VMEM((GGRP * CHUNK, 32), jnp.float32),
                       pltpu.VMEM((GGRP * CHUNK, 16), jnp.float32),
                       pltpu.SemaphoreType.DMA),
    )


def _edge1_body(edat_ref, w0, w1, w2, w3r, r16, t32, vrep, srep,
                v0, v1, v2, v3, out_ref, wsh_ref):
    i = pl.program_id(0)
    ps = edat_ref[:, 0:3]
    pd = edat_ref[:, 32:35]
    ev = ps - pd
    xj = edat_ref[:, 16:32]
    r2 = jnp.sum(ev * ev, axis=1, keepdims=True) + 1e-12
    inv = lax.rsqrt(r2)
    r = r2 * inv
    sh1 = SH1C * ev * inv
    cent = lax.broadcasted_iota(jnp.int32, (BE, NBASIS), 1).astype(jnp.float32) * CSTEP
    d = r - cent
    hb = jnp.exp(-GAMMA * d * d).astype(jnp.bfloat16)
    h1 = _swish16(jnp.dot(hb, w0[...], preferred_element_type=jnp.float32))
    h2 = _swish16(jnp.dot(h1, w1[...], preferred_element_type=jnp.float32))
    h3 = _swish16(jnp.dot(h2, w2[...], preferred_element_type=jnp.float32))
    xrep = jnp.dot(xj.astype(jnp.bfloat16), r16[...],
                   preferred_element_type=jnp.float32)
    hrep = jnp.dot(h3, t32[...], preferred_element_type=jnp.float32)
    acc = jnp.dot((xrep * hrep).astype(jnp.bfloat16), w3r[...],
                  preferred_element_type=jnp.float32)
    msc = acc[:, 0:24] * (SH0 * 0.25)
    tb = acc[:, 24:32] * 0.25
    l1 = (jnp.dot(tb, vrep[...], preferred_element_type=jnp.float32) *
          jnp.dot(sh1, srep[...], preferred_element_type=jnp.float32))
    msg = jnp.concatenate([msc, l1], axis=1)
    gid = i * BE + lax.broadcasted_iota(jnp.int32, (BE, 1), 0)
    out_ref[:, 0:48] = jnp.where(gid < E, msg, 0.0)
    g1 = _swish16(jnp.dot(hb, v0[...], preferred_element_type=jnp.float32))
    g2 = _swish16(jnp.dot(g1, v1[...], preferred_element_type=jnp.float32))
    g3 = _swish16(jnp.dot(g2, v2[...], preferred_element_type=jnp.float32))
    w2e = jnp.dot(g3, v3[...], preferred_element_type=jnp.float32)
    wsh_ref[...] = jnp.concatenate(
        [w2e, sh1, jnp.zeros((BE, 5), jnp.float32)], axis=1)


_edge1 = pl.pallas_call(
    _edge1_body,
    grid=(GE,),
    in_specs=[
        pl.BlockSpec((BE, 128), lambda i: (i, 0)),
        pl.BlockSpec((16, 64), lambda i: (0, 0)),
        pl.BlockSpec((64, 64), lambda i: (0, 0)),
        pl.BlockSpec((64, 32), lambda i: (0, 0)),
        pl.BlockSpec((512, 32), lambda i: (0, 0)),
        pl.BlockSpec((16, 512), lambda i: (0, 0)),
        pl.BlockSpec((32, 512), lambda i: (0, 0)),
        pl.BlockSpec((8, 24), lambda i: (0, 0)),
        pl.BlockSpec((3, 24), lambda i: (0, 0)),
        pl.BlockSpec((16, 64), lambda i: (0, 0)),
        pl.BlockSpec((64, 64), lambda i: (0, 0)),
        pl.BlockSpec((64, 32), lambda i: (0, 0)),
        pl.BlockSpec((32, 24), lambda i: (0, 0)),
    ],
    out_specs=(pl.BlockSpec((BE, 128), lambda i: (i, 0)),
               pl.BlockSpec((BE, 32), lambda i: (i, 0))),
    out_shape=(jax.ShapeDtypeStruct((EP, 128), jnp.float32),
               jax.ShapeDtypeStruct((EP, 32), jnp.float32)),
)


def _scatter_body(dst2, msg_hbm, zeros_hbm, agg_hbm, shared, idx_v, msg_v):
    c = lax.axis_index("c")
    s = lax.axis_index("s")
    pltpu.sync_copy(zeros_hbm.at[pl.ds(s * ROWS_PT, ROWS_PT)],
                    shared.at[pl.ds(s * ROWS_PT, ROWS_PT)])
    cbase = c * (NCHUNKS // 2) + s * CPT
    pltpu.sync_copy(dst2.at[pl.ds(cbase, CPT)], idx_v)
    plsc.subcore_barrier()

    def body(g, _):
        ebase = (cbase + g * GRP) * CHUNK
        pltpu.sync_copy(msg_hbm.at[pl.ds(ebase, GRP * CHUNK), pl.ds(0, 48)],
                        msg_v)
        for j in range(GRP):
            pltpu.sync_copy(msg_v.at[pl.ds(j * CHUNK, CHUNK)],
                            shared.at[idx_v.at[g * GRP + j]], add=True)
        return ()

    lax.fori_loop(0, NGRP, body, ())
    plsc.subcore_barrier()
    pltpu.sync_copy(shared.at[pl.ds(s * ROWS_PT, ROWS_PT)],
                    agg_hbm.at[c, pl.ds(s * ROWS_PT, ROWS_PT)])


@functools.cache
def _scatter_kernel():
    return pl.kernel(
        _scatter_body,
        out_type=jax.ShapeDtypeStruct((2, NP, 48), jnp.float32),
        mesh=plsc.VectorSubcoreMesh(core_axis_name="c", subcore_axis_name="s"),
        compiler_params=pltpu.CompilerParams(use_tc_tiling_on_sc=False),
        scratch_types=(pltpu.VMEM_SHARED((NP, 48), jnp.float32),
                       pltpu.VMEM((CPT, CHUNK), jnp.int32),
                       pltpu.VMEM((GRP * CHUNK, 48), jnp.float32)),
    )


BN = 1024
GN = NP // BN


def _node_body(ts_ref, agg_ref, si1, si2, grep, hg_ref, s2_ref):
    i = pl.program_id(0)
    ts = ts_ref[...]
    h0 = ts[:, 16:32]
    agg = agg_ref[0] + agg_ref[1]
    s24 = jnp.dot(h0, si1[...], preferred_element_type=jnp.float32)
    h24 = C * (s24 + agg[:, 0:24])
    scp = h24[:, 0:16]
    sc = scp * jax.nn.sigmoid(scp)
    g = jax.nn.sigmoid(h24[:, 16:24])
    l1g = agg[:, 24:48] * jnp.dot(g, grep[...],
                                  preferred_element_type=jnp.float32)
    hg_ref[...] = jnp.concatenate(
        [sc, l1g, jnp.zeros((BN, 8), jnp.float32)], axis=1)
    t = jnp.dot(sc, si2[...], preferred_element_type=jnp.float32)

    @pl.when(i == 0)
    def _():
        s2_ref[...] = jnp.zeros((1, 1), jnp.float32)

    s2_ref[...] += jnp.sum(t).reshape(1, 1)


_node = pl.pallas_call(
    _node_body,
    grid=(GN,),
    in_specs=[
        pl.BlockSpec((BN, 32), lambda i: (i, 0)),
        pl.BlockSpec((2, BN, 48), lambda i: (0, i, 0)),
        pl.BlockSpec((16, 24), lambda i: (0, 0)),
        pl.BlockSpec((16, 1), lambda i: (0, 0)),
        pl.BlockSpec((8, 24), lambda i: (0, 0)),
    ],
    out_specs=(pl.BlockSpec((BN, 48), lambda i: (i, 0)),
               pl.BlockSpec((1, 1), lambda i: (0, 0))),
    out_shape=(jax.ShapeDtypeStruct((NP, 48), jnp.float32),
               jax.ShapeDtypeStruct((1, 1), jnp.float32)),
)


def _gather2_body(src2, hg_hbm, ghg_hbm, idx_v, o_v, sem):
    c = lax.axis_index("c")
    s = lax.axis_index("s")
    cbase = (s * 2 + c) * CPT
    ngrp = CPT // GGRP
    pltpu.sync_copy(src2.at[pl.ds(cbase, CPTMAX)], idx_v)

    def body(g, _):
        descs = []
        for j in range(GGRP):
            descs.append(pltpu.async_copy(
                hg_hbm.at[idx_v.at[g * GGRP + j]],
                o_v.at[pl.ds(j * CHUNK, CHUNK)], sem))
        for d in descs:
            d.wait()
        ebase = (cbase + g * GGRP) * CHUNK
        pltpu.sync_copy(o_v,
                        ghg_hbm.at[pl.ds(ebase, GGRP * CHUNK), pl.ds(0, 48)])
        return ()

    lax.fori_loop(0, ngrp, body, ())


@functools.cache
def _gather2_kernel():
    return pl.kernel(
        _gather2_body,
        out_type=jax.ShapeDtypeStruct((EP, 128), jnp.float32),
        mesh=plsc.VectorSubcoreMesh(core_axis_name="c", subcore_axis_name="s"),
        compiler_params=pltpu.CompilerParams(use_tc_tiling_on_sc=False),
        scratch_types=(pltpu.VMEM((CPTMAX, CHUNK), jnp.int32),
                       pltpu.VMEM((GGRP * CHUNK, 48), jnp.float32),
                       pltpu.SemaphoreType.DMA),
    )


def _edge2_body(wsh_ref, ghg_ref, sexp, msum,
                ones24, out_ref):
    i = pl.program_id(0)
    ws = wsh_ref[...]
    w2e = ws[:, 0:24]
    sh1 = ws[:, 24:27]
    hg = ghg_ref[:, 0:48]
    xsc = hg[:, 0:16]
    sh1rep = jnp.dot(sh1, sexp[...], preferred_element_type=jnp.float32)
    dts = jnp.dot(hg * sh1rep, msum[...], preferred_element_type=jnp.float32)
    cat = jnp.concatenate([xsc * SH0, dts * INV_SQRT3], axis=1)
    m0 = jnp.dot(w2e * cat, ones24[...],
                 preferred_element_type=jnp.float32) * INV_SQRT24
    gid = i * BE + lax.broadcasted_iota(jnp.int32, (BE, 1), 0)
    m0 = jnp.where(gid < E, m0, 0.0)

    @pl.when(i == 0)
    def _():
        out_ref[...] = jnp.zeros((1, 1), jnp.float32)

    out_ref[...] += jnp.sum(m0).reshape(1, 1)


_edge2 = pl.pallas_call(
    _edge2_body,
    grid=(GE,),
    in_specs=[
        pl.BlockSpec((BE, 32), lambda i: (i, 0)),
        pl.BlockSpec((BE, 128), lambda i: (i, 0)),
        pl.BlockSpec((3, 48), lambda i: (0, 0)),
        pl.BlockSpec((48, 8), lambda i: (0, 0)),
        pl.BlockSpec((24, 1), lambda i: (0, 0)),
    ],
    out_specs=pl.BlockSpec((1, 1), lambda i: (0, 0)),
    out_shape=jax.ShapeDtypeStruct((1, 1), jnp.float32),
)


def kernel(z, pos, edge_index, emb, si1_W, rm1_w0, rm1_w1, rm1_w2, rm1_w3,
           si2_W, rm2_w0, rm2_w1, rm2_w2, rm2_w3):
    src = edge_index[0].astype(jnp.int32)
    dst = edge_index[1].astype(jnp.int32)
    src2 = jnp.pad(jnp.pad(src, (0, EP - E)).reshape(NCHUNKS, CHUNK),
                   ((0, 64), (0, 0)))
    dst2 = jnp.pad(jnp.pad(dst, (0, EP - E)).reshape(NCHUNKS, CHUNK),
                   ((0, 64), (0, 0)))
    posp = jnp.pad(pos.astype(jnp.float32), ((0, NP - N), (0, 13)))
    z2 = jnp.pad(z.astype(jnp.int32), (0, NP - N),
                 constant_values=100).reshape(NP, 1)

    w0s = (rm1_w0 * 0.25).astype(jnp.bfloat16)
    w1s = (rm1_w1 * 0.125).astype(jnp.bfloat16)
    w2s = (rm1_w2 * 0.125).astype(jnp.bfloat16)
    w3p = (rm1_w3 * float(1.0 / np.sqrt(32.0)))[:, _PERM]
    w3r = w3p.reshape(32, 16, 32).transpose(1, 0, 2).reshape(512, 32)
    w3r = w3r.astype(jnp.bfloat16)
    r16 = jnp.asarray(_R16).astype(jnp.bfloat16)
    t32 = jnp.asarray(_T32).astype(jnp.bfloat16)
    vrep = jnp.asarray(_VREP)
    srep = jnp.asarray(_SREP)
    sexp = jnp.asarray(_SEXP48)
    msum = jnp.asarray(_MSUM48)
    ones24 = jnp.asarray(_ONES24)
    v0s = (rm2_w0 * 0.25).astype(jnp.bfloat16)
    v1s = (rm2_w1 * 0.125).astype(jnp.bfloat16)
    v2s = (rm2_w2 * 0.125).astype(jnp.bfloat16)
    v3s = (rm2_w3 * float(1.0 / np.sqrt(32.0))).astype(jnp.bfloat16)
    si1s = si1_W * 0.25
    si2s = si2_W * 0.25
    zeros48 = jnp.zeros((NP, 48), jnp.float32)

    ts = _embed(z2, emb, posp)
    edat = _gather1_kernel()(src2, dst2, ts, posp)
    msg, wsh = _edge1(edat, w0s, w1s, w2s, w3r, r16, t32, vrep, srep,
                      v0s, v1s, v2s, v3s)
    agg = _scatter_kernel()(dst2, msg, zeros48)
    hg, s2 = _node(ts, agg, si1s, si2s, vrep)
    ghg = _gather2_kernel()(src2, hg)
    m0s = _edge2(wsh, ghg, sexp, msum, ones24)
    return C * (s2 + m0s)


# speedup vs baseline: 1.0278x; 1.0278x over previous
import functools

import numpy as np
import jax
import jax.numpy as jnp
from jax import lax
from jax.experimental import pallas as pl
from jax.experimental.pallas import tpu as pltpu
from jax.experimental.pallas import tpu_sc as plsc

N = 10000
E = 160000
NP = 10240
EP = 163840
MUL0 = 16
MUL1 = 8
CUTOFF = 10.0
NBASIS = 16
NUM_NEIGHBORS = 20
C = float(np.sqrt(0.5))
GAMMA = float(NBASIS / CUTOFF)
SH0 = float(1.0 / np.sqrt(NUM_NEIGHBORS))
SH1C = float(np.sqrt(3.0) / np.sqrt(NUM_NEIGHBORS))
INV_SQRT3 = float(1.0 / np.sqrt(3.0))
INV_SQRT24 = float(1.0 / np.sqrt(MUL0 + MUL1))

CHUNK = 128
NCHUNKS = EP // CHUNK
NTILES = 32
CPT = NCHUNKS // NTILES
GRP = 10
NGRP = CPT // GRP
ROWS_PT = NP // 16
GGRP = 10
G1_C0, G1_C1 = 40, 40
G2_C0, G2_C1 = 40, 40
CPTMAX = 40

BE = 2048
GE = EP // BE

CSTEP = float(CUTOFF / (NBASIS - 1))

_PERM = np.concatenate(
    [np.concatenate([np.arange(u * 24, (u + 1) * 24),
                     384 + np.arange(u * 8, (u + 1) * 8)]) for u in range(16)]
)


def _swish(x):
    return x * jax.nn.sigmoid(x)


def _swish16(x):
    return (x * jax.nn.sigmoid(x)).astype(jnp.bfloat16)


def _np_r16():
    r = np.zeros((16, 512), np.float32)
    for u in range(16):
        r[u, u * 32:(u + 1) * 32] = 1.0
    return r


def _np_t32():
    t = np.zeros((32, 512), np.float32)
    for k in range(32):
        t[k, k::32] = 1.0
    return t


def _np_vrep():
    a = np.zeros((8, 24), np.float32)
    for v in range(8):
        for m in range(3):
            a[v, 8 * m + v] = 1.0
    return a


def _np_srep():
    a = np.zeros((3, 24), np.float32)
    for m in range(3):
        a[m, 8 * m:8 * m + 8] = 1.0
    return a


def _np_sexp48():
    a = np.zeros((3, 48), np.float32)
    for m in range(3):
        a[m, 16 + 8 * m:24 + 8 * m] = 1.0
    return a


def _np_msum48():
    a = np.zeros((48, 8), np.float32)
    for m in range(3):
        for v in range(8):
            a[16 + 8 * m + v, v] = 1.0
    return a


_R16 = _np_r16()
_T32 = _np_t32()
_VREP = _np_vrep()
_SREP = _np_srep()
_SEXP48 = _np_sexp48()
_MSUM48 = _np_msum48()
_ONES24 = np.ones((24, 1), np.float32)


def _embed_body(z_ref, emb_ref, posp_ref, ts_ref):
    z = z_ref[...]
    k = lax.broadcasted_iota(jnp.int32, (NP, 100), 1)
    oh = (z == k).astype(jnp.float32)
    h0 = jnp.dot(oh, emb_ref[...], preferred_element_type=jnp.float32)
    ts_ref[...] = jnp.concatenate([posp_ref[...], h0], axis=1)


_embed = pl.pallas_call(
    _embed_body,
    out_shape=jax.ShapeDtypeStruct((NP, 32), jnp.float32),
)


def _gather1_body(src2, dst2, ts_hbm, posp_hbm, edat_hbm,
                  idxs_v, idxd_v, osrc_v, odst_v, sem):
    c = lax.axis_index("c")
    s = lax.axis_index("s")
    cbase = (s * 2 + c) * CPT
    ngrp = CPT // GGRP
    pltpu.sync_copy(src2.at[pl.ds(cbase, CPTMAX)], idxs_v)
    pltpu.sync_copy(dst2.at[pl.ds(cbase, CPTMAX)], idxd_v)

    def body(g, _):
        descs = []
        for j in range(GGRP):
            jj = g * GGRP + j
            descs.append(pltpu.async_copy(
                ts_hbm.at[idxs_v.at[jj]],
                osrc_v.at[pl.ds(j * CHUNK, CHUNK)], sem))
            descs.append(pltpu.async_copy(
                posp_hbm.at[idxd_v.at[jj]],
                odst_v.at[pl.ds(j * CHUNK, CHUNK)], sem))
        for d in descs:
            d.wait()
        ebase = (cbase + g * GGRP) * CHUNK
        pltpu.sync_copy(osrc_v,
                        edat_hbm.at[pl.ds(ebase, GGRP * CHUNK), pl.ds(0, 32)])
        pltpu.sync_copy(odst_v,
                        edat_hbm.at[pl.ds(ebase, GGRP * CHUNK), pl.ds(32, 16)])
        return ()

    lax.fori_loop(0, ngrp, body, ())


@functools.cache
def _gather1_kernel():
    return pl.kernel(
        _gather1_body,
        out_type=jax.ShapeDtypeStruct((EP, 128), jnp.float32),
        mesh=plsc.VectorSubcoreMesh(core_axis_name="c", subcore_axis_name="s"),
        compiler_params=pltpu.CompilerParams(use_tc_tiling_on_sc=False),
        scratch_types=(pltpu.VMEM((CPTMAX, CHUNK), jnp.int32),
                       pltpu.VMEM((CPTMAX, CHUNK), jnp.int32),
                       pltpu.VMEM((GGRP * CHUNK, 32), jnp.float32),
                       pltpu.VMEM((GGRP * CHUNK, 16), jnp.float32),
                       pltpu.SemaphoreType.DMA),
    )


def _edge1_body(edat_ref, w0, w1, w2, w3r, r16, t32, vrep, srep,
                v0, v1, v2, v3, out_ref, wsh_ref):
    i = pl.program_id(0)
    ps = edat_ref[:, 0:3]
    pd = edat_ref[:, 32:35]
    ev = ps - pd
    xj = edat_ref[:, 16:32]
    r2 = jnp.sum(ev * ev, axis=1, keepdims=True) + 1e-12
    inv = lax.rsqrt(r2)
    r = r2 * inv
    sh1 = SH1C * ev * inv
    cent = lax.broadcasted_iota(jnp.int32, (BE, NBASIS), 1).astype(jnp.float32) * CSTEP
    d = r - cent
    hb = jnp.exp(-GAMMA * d * d).astype(jnp.bfloat16)
    h1 = _swish16(jnp.dot(hb, w0[...], preferred_element_type=jnp.float32))
    h2 = _swish16(jnp.dot(h1, w1[...], preferred_element_type=jnp.float32))
    h3 = _swish16(jnp.dot(h2, w2[...], preferred_element_type=jnp.float32))
    xrep = jnp.dot(xj.astype(jnp.bfloat16), r16[...],
                   preferred_element_type=jnp.float32)
    hrep = jnp.dot(h3, t32[...], preferred_element_type=jnp.float32)
    acc = jnp.dot((xrep * hrep).astype(jnp.bfloat16), w3r[...],
                  preferred_element_type=jnp.float32)
    msc = acc[:, 0:24] * (SH0 * 0.25)
    tb = acc[:, 24:32] * 0.25
    l1 = (jnp.dot(tb, vrep[...], preferred_element_type=jnp.float32) *
          jnp.dot(sh1, srep[...], preferred_element_type=jnp.float32))
    msg = jnp.concatenate([msc, l1], axis=1)
    gid = i * BE + lax.broadcasted_iota(jnp.int32, (BE, 1), 0)
    out_ref[:, 0:48] = jnp.where(gid < E, msg, 0.0)
    g1 = _swish16(jnp.dot(hb, v0[...], preferred_element_type=jnp.float32))
    g2 = _swish16(jnp.dot(g1, v1[...], preferred_element_type=jnp.float32))
    g3 = _swish16(jnp.dot(g2, v2[...], preferred_element_type=jnp.float32))
    w2e = jnp.dot(g3, v3[...], preferred_element_type=jnp.float32)
    wsh_ref[...] = jnp.concatenate(
        [w2e, sh1, jnp.zeros((BE, 5), jnp.float32)], axis=1)


_edge1 = pl.pallas_call(
    _edge1_body,
    grid=(GE,),
    in_specs=[
        pl.BlockSpec((BE, 128), lambda i: (i, 0)),
        pl.BlockSpec((16, 64), lambda i: (0, 0)),
        pl.BlockSpec((64, 64), lambda i: (0, 0)),
        pl.BlockSpec((64, 32), lambda i: (0, 0)),
        pl.BlockSpec((512, 32), lambda i: (0, 0)),
        pl.BlockSpec((16, 512), lambda i: (0, 0)),
        pl.BlockSpec((32, 512), lambda i: (0, 0)),
        pl.BlockSpec((8, 24), lambda i: (0, 0)),
        pl.BlockSpec((3, 24), lambda i: (0, 0)),
        pl.BlockSpec((16, 64), lambda i: (0, 0)),
        pl.BlockSpec((64, 64), lambda i: (0, 0)),
        pl.BlockSpec((64, 32), lambda i: (0, 0)),
        pl.BlockSpec((32, 24), lambda i: (0, 0)),
    ],
    out_specs=(pl.BlockSpec((BE, 128), lambda i: (i, 0)),
               pl.BlockSpec((BE, 32), lambda i: (i, 0))),
    out_shape=(jax.ShapeDtypeStruct((EP, 128), jnp.float32),
               jax.ShapeDtypeStruct((EP, 32), jnp.float32)),
)


def _scatter_body(dst2, msg_hbm, zeros_hbm, agg_hbm, shared, idx_v, msg_v):
    c = lax.axis_index("c")
    s = lax.axis_index("s")
    pltpu.sync_copy(zeros_hbm.at[pl.ds(s * ROWS_PT, ROWS_PT)],
                    shared.at[pl.ds(s * ROWS_PT, ROWS_PT)])
    cbase = c * (NCHUNKS // 2) + s * CPT
    pltpu.sync_copy(dst2.at[pl.ds(cbase, CPT)], idx_v)
    plsc.subcore_barrier()

    def body(g, _):
        ebase = (cbase + g * GRP) * CHUNK
        pltpu.sync_copy(msg_hbm.at[pl.ds(ebase, GRP * CHUNK), pl.ds(0, 48)],
                        msg_v)
        for j in range(GRP):
            pltpu.sync_copy(msg_v.at[pl.ds(j * CHUNK, CHUNK)],
                            shared.at[idx_v.at[g * GRP + j]], add=True)
        return ()

    lax.fori_loop(0, NGRP, body, ())
    plsc.subcore_barrier()
    pltpu.sync_copy(shared.at[pl.ds(s * ROWS_PT, ROWS_PT)],
                    agg_hbm.at[c, pl.ds(s * ROWS_PT, ROWS_PT)])


@functools.cache
def _scatter_kernel():
    return pl.kernel(
        _scatter_body,
        out_type=jax.ShapeDtypeStruct((2, NP, 48), jnp.float32),
        mesh=plsc.VectorSubcoreMesh(core_axis_name="c", subcore_axis_name="s"),
        compiler_params=pltpu.CompilerParams(use_tc_tiling_on_sc=False),
        scratch_types=(pltpu.VMEM_SHARED((NP, 48), jnp.float32),
                       pltpu.VMEM((CPT, CHUNK), jnp.int32),
                       pltpu.VMEM((GRP * CHUNK, 48), jnp.float32)),
    )


BN = 1024
GN = NP // BN


def _node_body(ts_ref, agg_ref, si1, si2, grep, hg_ref, s2_ref):
    i = pl.program_id(0)
    ts = ts_ref[...]
    h0 = ts[:, 16:32]
    agg = agg_ref[0] + agg_ref[1]
    s24 = jnp.dot(h0, si1[...], preferred_element_type=jnp.float32)
    h24 = C * (s24 + agg[:, 0:24])
    scp = h24[:, 0:16]
    sc = scp * jax.nn.sigmoid(scp)
    g = jax.nn.sigmoid(h24[:, 16:24])
    l1g = agg[:, 24:48] * jnp.dot(g, grep[...],
                                  preferred_element_type=jnp.float32)
    hg_ref[...] = jnp.concatenate(
        [sc, l1g, jnp.zeros((BN, 8), jnp.float32)], axis=1)
    t = jnp.dot(sc, si2[...], preferred_element_type=jnp.float32)

    @pl.when(i == 0)
    def _():
        s2_ref[...] = jnp.zeros((1, 1), jnp.float32)

    s2_ref[...] += jnp.sum(t).reshape(1, 1)


_node = pl.pallas_call(
    _node_body,
    grid=(GN,),
    in_specs=[
        pl.BlockSpec((BN, 32), lambda i: (i, 0)),
        pl.BlockSpec((2, BN, 48), lambda i: (0, i, 0)),
        pl.BlockSpec((16, 24), lambda i: (0, 0)),
        pl.BlockSpec((16, 1), lambda i: (0, 0)),
        pl.BlockSpec((8, 24), lambda i: (0, 0)),
    ],
    out_specs=(pl.BlockSpec((BN, 48), lambda i: (i, 0)),
               pl.BlockSpec((1, 1), lambda i: (0, 0))),
    out_shape=(jax.ShapeDtypeStruct((NP, 48), jnp.float32),
               jax.ShapeDtypeStruct((1, 1), jnp.float32)),
)


def _gather2_body(src2, hg_hbm, ghg_hbm, idx_v, o_v, sem):
    c = lax.axis_index("c")
    s = lax.axis_index("s")
    cbase = (s * 2 + c) * CPT
    ngrp = CPT // GGRP
    pltpu.sync_copy(src2.at[pl.ds(cbase, CPTMAX)], idx_v)

    def body(g, _):
        descs = []
        for j in range(GGRP):
            descs.append(pltpu.async_copy(
                hg_hbm.at[idx_v.at[g * GGRP + j]],
                o_v.at[pl.ds(j * CHUNK, CHUNK)], sem))
        for d in descs:
            d.wait()
        ebase = (cbase + g * GGRP) * CHUNK
        pltpu.sync_copy(o_v,
                        ghg_hbm.at[pl.ds(ebase, GGRP * CHUNK), pl.ds(0, 48)])
        return ()

    lax.fori_loop(0, ngrp, body, ())


@functools.cache
def _gather2_kernel():
    return pl.kernel(
        _gather2_body,
        out_type=jax.ShapeDtypeStruct((EP, 128), jnp.float32),
        mesh=plsc.VectorSubcoreMesh(core_axis_name="c", subcore_axis_name="s"),
        compiler_params=pltpu.CompilerParams(use_tc_tiling_on_sc=False),
        scratch_types=(pltpu.VMEM((CPTMAX, CHUNK), jnp.int32),
                       pltpu.VMEM((GGRP * CHUNK, 48), jnp.float32),
                       pltpu.SemaphoreType.DMA),
    )


def _edge2_body(wsh_ref, ghg_ref, sexp, msum,
                ones24, out_ref):
    i = pl.program_id(0)
    ws = wsh_ref[...]
    w2e = ws[:, 0:24]
    sh1 = ws[:, 24:27]
    hg = ghg_ref[:, 0:48]
    xsc = hg[:, 0:16]
    sh1rep = jnp.dot(sh1, sexp[...], preferred_element_type=jnp.float32)
    dts = jnp.dot(hg * sh1rep, msum[...], preferred_element_type=jnp.float32)
    cat = jnp.concatenate([xsc * SH0, dts * INV_SQRT3], axis=1)
    m0 = jnp.dot(w2e * cat, ones24[...],
                 preferred_element_type=jnp.float32) * INV_SQRT24
    gid = i * BE + lax.broadcasted_iota(jnp.int32, (BE, 1), 0)
    m0 = jnp.where(gid < E, m0, 0.0)

    @pl.when(i == 0)
    def _():
        out_ref[...] = jnp.zeros((1, 1), jnp.float32)

    out_ref[...] += jnp.sum(m0).reshape(1, 1)


_edge2 = pl.pallas_call(
    _edge2_body,
    grid=(GE,),
    in_specs=[
        pl.BlockSpec((BE, 32), lambda i: (i, 0)),
        pl.BlockSpec((BE, 128), lambda i: (i, 0)),
        pl.BlockSpec((3, 48), lambda i: (0, 0)),
        pl.BlockSpec((48, 8), lambda i: (0, 0)),
        pl.BlockSpec((24, 1), lambda i: (0, 0)),
    ],
    out_specs=pl.BlockSpec((1, 1), lambda i: (0, 0)),
    out_shape=jax.ShapeDtypeStruct((1, 1), jnp.float32),
)


def kernel(z, pos, edge_index, emb, si1_W, rm1_w0, rm1_w1, rm1_w2, rm1_w3,
           si2_W, rm2_w0, rm2_w1, rm2_w2, rm2_w3):
    src = edge_index[0].astype(jnp.int32)
    dst = edge_index[1].astype(jnp.int32)
    src2 = jnp.pad(src, (0, EP - E)).reshape(NCHUNKS, CHUNK)
    dst2 = jnp.pad(dst, (0, EP - E)).reshape(NCHUNKS, CHUNK)
    posp = jnp.pad(pos.astype(jnp.float32), ((0, NP - N), (0, 13)))
    z2 = jnp.pad(z.astype(jnp.int32), (0, NP - N),
                 constant_values=100).reshape(NP, 1)

    w0s = (rm1_w0 * 0.25).astype(jnp.bfloat16)
    w1s = (rm1_w1 * 0.125).astype(jnp.bfloat16)
    w2s = (rm1_w2 * 0.125).astype(jnp.bfloat16)
    w3p = (rm1_w3 * float(1.0 / np.sqrt(32.0)))[:, _PERM]
    w3r = w3p.reshape(32, 16, 32).transpose(1, 0, 2).reshape(512, 32)
    w3r = w3r.astype(jnp.bfloat16)
    r16 = jnp.asarray(_R16).astype(jnp.bfloat16)
    t32 = jnp.asarray(_T32).astype(jnp.bfloat16)
    vrep = jnp.asarray(_VREP)
    srep = jnp.asarray(_SREP)
    sexp = jnp.asarray(_SEXP48)
    msum = jnp.asarray(_MSUM48)
    ones24 = jnp.asarray(_ONES24)
    v0s = (rm2_w0 * 0.25).astype(jnp.bfloat16)
    v1s = (rm2_w1 * 0.125).astype(jnp.bfloat16)
    v2s = (rm2_w2 * 0.125).astype(jnp.bfloat16)
    v3s = (rm2_w3 * float(1.0 / np.sqrt(32.0))).astype(jnp.bfloat16)
    si1s = si1_W * 0.25
    si2s = si2_W * 0.25
    zeros48 = jnp.zeros((NP, 48), jnp.float32)

    ts = _embed(z2, emb, posp)
    edat = _gather1_kernel()(src2, dst2, ts, posp)
    msg, wsh = _edge1(edat, w0s, w1s, w2s, w3r, r16, t32, vrep, srep,
                      v0s, v1s, v2s, v3s)
    agg = _scatter_kernel()(dst2, msg, zeros48)
    hg, s2 = _node(ts, agg, si1s, si2s, vrep)
    ghg = _gather2_kernel()(src2, hg)
    m0s = _edge2(wsh, ghg, sexp, msum, ones24)
    return C * (s2 + m0s)
